# baseline (speedup 1.0000x reference)
import jax
import jax.numpy as jnp
from jax import lax
from jax.experimental import pallas as pl
from jax.experimental.pallas import tpu as pltpu

_EPS = 1e-12
_BS = 2048
_RT = 8
_UNROLL = 256


def _body(x_ref, t_ref, g_ref, b_ref, o_ref):
    d = t_ref.shape[-1]
    bs = x_ref.shape[1]
    inv_d = 1.0 / d
    g = g_ref[...]
    b = b_ref[...]
    s0 = pl.program_id(0) * bs

    def step(i, carry):
        for u in range(_UNROLL):
            r0 = (i * _UNROLL + u) * _RT
            emb = x_ref[0, pl.ds(r0, _RT), :] + t_ref[pl.ds(s0 + r0, _RT), :]
            mean = jnp.sum(emb, axis=-1, keepdims=True) * inv_d
            var = jnp.sum(emb * emb, axis=-1, keepdims=True) * inv_d - mean * mean
            inv = lax.rsqrt(var + _EPS)
            o_ref[0, pl.ds(r0, _RT), :] = (emb - mean) * (inv * g) + b
        return carry

    lax.fori_loop(0, bs // (_RT * _UNROLL), step, 0)


def kernel(x, table, gamma, beta):
    B, S, D = x.shape
    bs = _BS if S % _BS == 0 else S
    grid = (S // bs, B)
    return pl.pallas_call(
        _body,
        grid=grid,
        in_specs=[
            pl.BlockSpec((1, bs, D), lambda i, b: (b, i, 0)),
            pl.BlockSpec((S, D), lambda i, b: (0, 0)),
            pl.BlockSpec((1, D), lambda i, b: (0, 0)),
            pl.BlockSpec((1, D), lambda i, b: (0, 0)),
        ],
        out_specs=pl.BlockSpec((1, bs, D), lambda i, b: (b, i, 0)),
        out_shape=jax.ShapeDtypeStruct((B, S, D), x.dtype),
        compiler_params=pltpu.CompilerParams(
            dimension_semantics=("arbitrary", "arbitrary"),
        ),
    )(x, table[:S], gamma.reshape(1, D), beta.reshape(1, D))
